# Initial kernel scaffold; baseline (speedup 1.0000x reference)
import jax
import jax.numpy as jnp
from jax.experimental import pallas as pl


def kernel(x, edge_index, edge_weight, feat_mask, W1, b1, W2, b2):
    raise NotImplementedError("write your pallas kernel here")


# speedup vs baseline: 2.5252x; 2.5252x over previous
import functools

import jax
import jax.numpy as jnp
from jax.experimental import pallas as pl
from jax.experimental.pallas import tpu as pltpu

N = 10000
F_IN = 128
H = 16
ROW_BLK = 2000


def _dense_a(x_ref, fm_ref, w1_ref, fm_out, h1_out):
    fm = jax.nn.sigmoid(fm_ref[...])
    fm_out[...] = fm
    xm = x_ref[...] * fm
    h1_out[...] = jnp.dot(xm, w1_ref[...], preferred_element_type=jnp.float32)


def _dense_c(deg_ref, h1_ref, dis_out, dis2_out, g1_out):
    dis = jax.lax.rsqrt(deg_ref[...])
    dis_out[...] = dis
    dis2_out[...] = dis * dis
    g1_out[...] = dis * h1_ref[...]


def _dense_e(acc1_ref, dis_ref, dis2_ref, h1_ref, b1_ref, w2_ref,
             h2_out, g2_out):
    acc = acc1_ref[0] + acc1_ref[1]
    out1 = jax.nn.relu(dis_ref[...] * acc + dis2_ref[...] * h1_ref[...]
                       + b1_ref[...])
    h2 = jnp.dot(out1, w2_ref[...], preferred_element_type=jnp.float32)
    h2_out[...] = h2
    g2_out[...] = dis_ref[...] * h2


def _dense_g(acc2_ref, dis_ref, dis2_ref, h2_ref, b2_ref, out_ref):
    acc = acc2_ref[0] + acc2_ref[1]
    o = dis_ref[...] * acc + dis2_ref[...] * h2_ref[...] + b2_ref[...]
    mask = jax.lax.broadcasted_iota(jnp.int32, o.shape, 1) < 7
    neg = jnp.full_like(o, -jnp.inf)
    om = jnp.where(mask, o, neg)
    m = jnp.max(om, axis=1, keepdims=True)
    ex = jnp.where(mask, jnp.exp(o - m), jnp.zeros_like(o))
    lse = jnp.log(jnp.sum(ex, axis=1, keepdims=True))
    out_ref[...] = o - m - lse


def _row_spec(width):
    return pl.BlockSpec((ROW_BLK, width), lambda i: (i, 0))


def _acc_spec(width):
    return pl.BlockSpec((2, ROW_BLK, width), lambda i: (0, i, 0))


def _full_spec(shape):
    return pl.BlockSpec(shape, lambda i: tuple(0 for _ in shape))


def _segment_add(vals, col, width):
    acc = jax.ops.segment_sum(vals, col, num_segments=N)
    return jnp.stack([acc, jnp.zeros_like(acc)])


def kernel(x, edge_index, edge_weight, feat_mask, W1, b1, W2, b2):
    row = edge_index[0].astype(jnp.int32)
    col = edge_index[1].astype(jnp.int32)
    w = edge_weight

    grid = (N // ROW_BLK,)

    fm, h1 = pl.pallas_call(
        _dense_a,
        grid=grid,
        in_specs=[_row_spec(F_IN), _row_spec(F_IN), _full_spec((F_IN, H))],
        out_specs=[_row_spec(F_IN), _row_spec(H)],
        out_shape=[jax.ShapeDtypeStruct((N, F_IN), jnp.float32),
                   jax.ShapeDtypeStruct((N, H), jnp.float32)],
    )(x, feat_mask, W1)

    degp = _segment_add(w, col, 1)
    deg2d = (degp[0] + degp[1] + 1.0)[:, None]

    dis, dis2, g1 = pl.pallas_call(
        _dense_c,
        grid=grid,
        in_specs=[_row_spec(1), _row_spec(H)],
        out_specs=[_row_spec(1), _row_spec(1), _row_spec(H)],
        out_shape=[jax.ShapeDtypeStruct((N, 1), jnp.float32),
                   jax.ShapeDtypeStruct((N, 1), jnp.float32),
                   jax.ShapeDtypeStruct((N, H), jnp.float32)],
    )(deg2d, h1)

    acc1 = _segment_add(w[:, None] * g1[row], col, H)

    W2p = jnp.zeros((H, H), jnp.float32).at[:, :W2.shape[1]].set(W2)
    b1r = b1[None, :]
    b2p = jnp.zeros((1, H), jnp.float32).at[0, :b2.shape[0]].set(b2)

    h2, g2 = pl.pallas_call(
        _dense_e,
        grid=grid,
        in_specs=[_acc_spec(H), _row_spec(1), _row_spec(1), _row_spec(H),
                  _full_spec((1, H)), _full_spec((H, H))],
        out_specs=[_row_spec(H), _row_spec(H)],
        out_shape=[jax.ShapeDtypeStruct((N, H), jnp.float32),
                   jax.ShapeDtypeStruct((N, H), jnp.float32)],
    )(acc1, dis, dis2, h1, b1r, W2p)

    acc2 = _segment_add(w[:, None] * g2[row], col, H)

    outp = pl.pallas_call(
        _dense_g,
        grid=grid,
        in_specs=[_acc_spec(H), _row_spec(1), _row_spec(1), _row_spec(H),
                  _full_spec((1, H))],
        out_specs=_row_spec(H),
        out_shape=jax.ShapeDtypeStruct((N, H), jnp.float32),
    )(acc2, dis, dis2, h2, b2p)

    return outp[:, :7], fm


# speedup vs baseline: 23.0059x; 9.1106x over previous
import jax
import jax.numpy as jnp
from jax import lax
from jax.experimental import pallas as pl
from jax.experimental.pallas import tpu as pltpu
from jax.experimental.pallas import tpu_sc as plsc

N = 10000
F_IN = 128
H = 16
E = 320000
ROW_BLK = 2000

NC = 2
NS = 16
GRP = 128
GPC = 16
CHUNK = GRP * GPC
NCHUNK = 5
EPT = CHUNK * NCHUNK
EP = EPT * NC * NS
NP = 10240
NPS = NP // NS


def _sc_deg_body(col_hbm, w_hbm, zeros_hbm, degp_hbm, col_v, w_v, deg_sh):
    cid = lax.axis_index("c")
    sid = lax.axis_index("s")
    gbase = (cid * NS + sid) * (EPT // GRP)
    nslice = pl.ds(sid * NPS, NPS)
    pltpu.sync_copy(zeros_hbm.at[nslice], deg_sh.at[nslice])
    plsc.subcore_barrier()

    def chunk(i, carry):
        pltpu.sync_copy(col_hbm.at[pl.ds(gbase + i * GPC, GPC)], col_v)
        pltpu.sync_copy(w_hbm.at[pl.ds((gbase + i * GPC) * GRP, CHUNK)], w_v)
        for j in range(GPC):
            pltpu.sync_copy(w_v.at[pl.ds(j * GRP, GRP)],
                            deg_sh.at[col_v.at[j]], add=True)
        return carry

    lax.fori_loop(0, NCHUNK, chunk, 0)
    plsc.subcore_barrier()
    pltpu.sync_copy(deg_sh.at[nslice], degp_hbm.at[cid, nslice])


def _sc_prop_body(row_hbm, col_hbm, w_hbm, g_hbm, zeros_hbm, accp_hbm,
                  row_v, col_v, w_v, msg_v, acc_sh):
    cid = lax.axis_index("c")
    sid = lax.axis_index("s")
    gbase = (cid * NS + sid) * (EPT // GRP)
    nslice = pl.ds(sid * NPS, NPS)
    pltpu.sync_copy(zeros_hbm.at[nslice], acc_sh.at[nslice])
    plsc.subcore_barrier()

    def chunk(i, carry):
        pltpu.sync_copy(row_hbm.at[pl.ds(gbase + i * GPC, GPC)], row_v)
        pltpu.sync_copy(col_hbm.at[pl.ds(gbase + i * GPC, GPC)], col_v)
        pltpu.sync_copy(w_hbm.at[pl.ds((gbase + i * GPC) * GRP, CHUNK)], w_v)
        for j in range(GPC):
            pltpu.sync_copy(g_hbm.at[row_v.at[j]],
                            msg_v.at[pl.ds(j * GRP, GRP)])

        def group(gi, c2):
            e0 = gi * 16
            wv = w_v[pl.ds(e0, 16)]
            for k in range(16):
                e = e0 + k
                msg_v[e] = msg_v[e] * wv[k]
            return c2

        lax.fori_loop(0, CHUNK // 16, group, 0)
        for j in range(GPC):
            pltpu.sync_copy(msg_v.at[pl.ds(j * GRP, GRP)],
                            acc_sh.at[col_v.at[j]], add=True)
        return carry

    lax.fori_loop(0, NCHUNK, chunk, 0)
    plsc.subcore_barrier()
    pltpu.sync_copy(acc_sh.at[nslice], accp_hbm.at[cid, nslice])


_SC_MESH = plsc.VectorSubcoreMesh(
    core_axis_name="c", subcore_axis_name="s", num_cores=NC, num_subcores=NS)

_deg_call = pl.kernel(
    _sc_deg_body,
    out_type=jax.ShapeDtypeStruct((NC, NP), jnp.float32),
    mesh=_SC_MESH,
    scratch_types=[
        pltpu.VMEM((GPC, GRP), jnp.int32),
        pltpu.VMEM((CHUNK,), jnp.float32),
        pltpu.VMEM_SHARED((NP,), jnp.float32),
    ],
)

_prop_call = pl.kernel(
    _sc_prop_body,
    out_type=jax.ShapeDtypeStruct((NC, NP, H), jnp.float32),
    mesh=_SC_MESH,
    compiler_params=pltpu.CompilerParams(use_tc_tiling_on_sc=False),
    scratch_types=[
        pltpu.VMEM((GPC, GRP), jnp.int32),
        pltpu.VMEM((GPC, GRP), jnp.int32),
        pltpu.VMEM((CHUNK,), jnp.float32),
        pltpu.VMEM((CHUNK, H), jnp.float32),
        pltpu.VMEM_SHARED((NP, H), jnp.float32),
    ],
)


def _dense_a(x_ref, fm_ref, w1_ref, fm_out, h1_out):
    fm = jax.nn.sigmoid(fm_ref[...])
    fm_out[...] = fm
    xm = x_ref[...] * fm
    h1_out[...] = jnp.dot(xm, w1_ref[...], preferred_element_type=jnp.float32)


def _dense_c(dega_ref, degb_ref, h1_ref, dis_out, dis2_out, g1_out):
    dis = jax.lax.rsqrt(dega_ref[...] + degb_ref[...] + 1.0)
    dis_out[...] = dis
    dis2_out[...] = dis * dis
    g1_out[...] = dis * h1_ref[...]


def _dense_e(acc1_ref, dis_ref, dis2_ref, h1_ref, b1_ref, w2_ref,
             h2_out, g2_out):
    acc = acc1_ref[0] + acc1_ref[1]
    out1 = jax.nn.relu(dis_ref[...] * acc + dis2_ref[...] * h1_ref[...]
                       + b1_ref[...])
    h2 = jnp.dot(out1, w2_ref[...], preferred_element_type=jnp.float32)
    h2_out[...] = h2
    g2_out[...] = dis_ref[...] * h2


def _dense_g(acc2_ref, dis_ref, dis2_ref, h2_ref, b2_ref, out_ref):
    acc = acc2_ref[0] + acc2_ref[1]
    o = dis_ref[...] * acc + dis2_ref[...] * h2_ref[...] + b2_ref[...]
    mask = jax.lax.broadcasted_iota(jnp.int32, o.shape, 1) < 7
    neg = jnp.full_like(o, -jnp.inf)
    om = jnp.where(mask, o, neg)
    m = jnp.max(om, axis=1, keepdims=True)
    ex = jnp.where(mask, jnp.exp(o - m), jnp.zeros_like(o))
    lse = jnp.log(jnp.sum(ex, axis=1, keepdims=True))
    out_ref[...] = o - m - lse


def _row_spec(width):
    return pl.BlockSpec((ROW_BLK, width), lambda i: (i, 0))


def _acc_spec(width):
    return pl.BlockSpec((2, ROW_BLK, width), lambda i: (0, i, 0))


def _full_spec(shape):
    return pl.BlockSpec(shape, lambda i: tuple(0 for _ in shape))


def kernel(x, edge_index, edge_weight, feat_mask, W1, b1, W2, b2):
    row = edge_index[0].astype(jnp.int32)
    col = edge_index[1].astype(jnp.int32)
    w = edge_weight.astype(jnp.float32)

    pad = EP - E
    row2d = jnp.concatenate([row, jnp.zeros((pad,), jnp.int32)]).reshape(
        EP // GRP, GRP)
    col2d = jnp.concatenate([col, jnp.zeros((pad,), jnp.int32)]).reshape(
        EP // GRP, GRP)
    wp = jnp.concatenate([w, jnp.zeros((pad,), jnp.float32)])
    zeros1 = jnp.zeros((NP,), jnp.float32)
    zeros2 = jnp.zeros((NP, H), jnp.float32)

    grid = (N // ROW_BLK,)

    fm, h1 = pl.pallas_call(
        _dense_a,
        grid=grid,
        in_specs=[_row_spec(F_IN), _row_spec(F_IN), _full_spec((F_IN, H))],
        out_specs=[_row_spec(F_IN), _row_spec(H)],
        out_shape=[jax.ShapeDtypeStruct((N, F_IN), jnp.float32),
                   jax.ShapeDtypeStruct((N, H), jnp.float32)],
    )(x, feat_mask, W1)

    degp = _deg_call(col2d, wp, zeros1)
    dega = degp[0, :N, None]
    degb = degp[1, :N, None]

    dis, dis2, g1 = pl.pallas_call(
        _dense_c,
        grid=grid,
        in_specs=[_row_spec(1), _row_spec(1), _row_spec(H)],
        out_specs=[_row_spec(1), _row_spec(1), _row_spec(H)],
        out_shape=[jax.ShapeDtypeStruct((N, 1), jnp.float32),
                   jax.ShapeDtypeStruct((N, 1), jnp.float32),
                   jax.ShapeDtypeStruct((N, H), jnp.float32)],
    )(dega, degb, h1)

    acc1 = _prop_call(row2d, col2d, wp, g1, zeros2)[:, :N, :]

    W2p = jnp.zeros((H, H), jnp.float32).at[:, :W2.shape[1]].set(W2)
    b1r = b1[None, :]
    b2p = jnp.zeros((1, H), jnp.float32).at[0, :b2.shape[0]].set(b2)

    h2, g2 = pl.pallas_call(
        _dense_e,
        grid=grid,
        in_specs=[_acc_spec(H), _row_spec(1), _row_spec(1), _row_spec(H),
                  _full_spec((1, H)), _full_spec((H, H))],
        out_specs=[_row_spec(H), _row_spec(H)],
        out_shape=[jax.ShapeDtypeStruct((N, H), jnp.float32),
                   jax.ShapeDtypeStruct((N, H), jnp.float32)],
    )(acc1, dis, dis2, h1, b1r, W2p)

    acc2 = _prop_call(row2d, col2d, wp, g2, zeros2)[:, :N, :]

    outp = pl.pallas_call(
        _dense_g,
        grid=grid,
        in_specs=[_acc_spec(H), _row_spec(1), _row_spec(1), _row_spec(H),
                  _full_spec((1, H))],
        out_specs=_row_spec(H),
        out_shape=jax.ShapeDtypeStruct((N, H), jnp.float32),
    )(acc2, dis, dis2, h2, b2p)

    return outp[:, :7], fm


# speedup vs baseline: 27.7115x; 1.2045x over previous
import jax
import jax.numpy as jnp
from jax import lax
from jax.experimental import pallas as pl
from jax.experimental.pallas import tpu as pltpu
from jax.experimental.pallas import tpu_sc as plsc

N = 10000
F_IN = 128
H = 16
E = 320000
ROW_BLK = 2000

NC = 2
NS = 16
GRP = 128
GPC = 16
CHUNK = GRP * GPC
NCHUNK = 5
EPT = CHUNK * NCHUNK
EP = EPT * NC * NS
NP = 10240
NPS = NP // NS


def _sc_deg_body(col_hbm, w_hbm, zeros_hbm, degp_hbm, col_v, w_v, deg_sh):
    cid = lax.axis_index("c")
    sid = lax.axis_index("s")
    gbase = (cid * NS + sid) * (EPT // GRP)
    nslice = pl.ds(sid * NPS, NPS)
    pltpu.sync_copy(zeros_hbm.at[nslice], deg_sh.at[nslice])
    plsc.subcore_barrier()

    def chunk(i, carry):
        ebase = (gbase + i * GPC) * GRP
        pltpu.sync_copy(col_hbm.at[pl.ds(ebase, CHUNK)], col_v)
        pltpu.sync_copy(w_hbm.at[pl.ds(ebase, CHUNK)], w_v)
        pltpu.sync_copy(w_v, deg_sh.at[col_v], add=True)
        return carry

    lax.fori_loop(0, NCHUNK, chunk, 0)
    plsc.subcore_barrier()
    pltpu.sync_copy(deg_sh.at[nslice], degp_hbm.at[cid, nslice])


def _sc_prop_body(row_hbm, col_hbm, w_hbm, g_hbm, zeros_hbm, accp_hbm,
                  row_v, col_v, w_v, msg_v, acc_sh):
    cid = lax.axis_index("c")
    sid = lax.axis_index("s")
    gbase = (cid * NS + sid) * (EPT // GRP)
    nslice = pl.ds(sid * NPS, NPS)
    pltpu.sync_copy(zeros_hbm.at[nslice], acc_sh.at[nslice])
    plsc.subcore_barrier()

    def chunk(i, carry):
        ebase = (gbase + i * GPC) * GRP
        pltpu.sync_copy(row_hbm.at[pl.ds(ebase, CHUNK)], row_v)
        pltpu.sync_copy(col_hbm.at[pl.ds(ebase, CHUNK)], col_v)
        pltpu.sync_copy(w_hbm.at[pl.ds(ebase, CHUNK)], w_v)
        pltpu.sync_copy(g_hbm.at[row_v], msg_v)

        def group(gi, c2):
            e0 = gi * 16
            wv = w_v[pl.ds(e0, 16)]
            for k in range(16):
                e = e0 + k
                msg_v[e] = msg_v[e] * wv[k]
            return c2

        lax.fori_loop(0, CHUNK // 16, group, 0)
        pltpu.sync_copy(msg_v, acc_sh.at[col_v], add=True)
        return carry

    lax.fori_loop(0, NCHUNK, chunk, 0)
    plsc.subcore_barrier()
    pltpu.sync_copy(acc_sh.at[nslice], accp_hbm.at[cid, nslice])


_SC_MESH = plsc.VectorSubcoreMesh(
    core_axis_name="c", subcore_axis_name="s", num_cores=NC, num_subcores=NS)

_deg_call = pl.kernel(
    _sc_deg_body,
    out_type=jax.ShapeDtypeStruct((NC, NP), jnp.float32),
    mesh=_SC_MESH,
    scratch_types=[
        pltpu.VMEM((CHUNK,), jnp.int32),
        pltpu.VMEM((CHUNK,), jnp.float32),
        pltpu.VMEM_SHARED((NP,), jnp.float32),
    ],
)

_prop_call = pl.kernel(
    _sc_prop_body,
    out_type=jax.ShapeDtypeStruct((NC, NP, H), jnp.float32),
    mesh=_SC_MESH,
    compiler_params=pltpu.CompilerParams(use_tc_tiling_on_sc=False),
    scratch_types=[
        pltpu.VMEM((CHUNK,), jnp.int32),
        pltpu.VMEM((CHUNK,), jnp.int32),
        pltpu.VMEM((CHUNK,), jnp.float32),
        pltpu.VMEM((CHUNK, H), jnp.float32),
        pltpu.VMEM_SHARED((NP, H), jnp.float32),
    ],
)


def _dense_a(x_ref, fm_ref, w1_ref, fm_out, h1_out):
    fm = jax.nn.sigmoid(fm_ref[...])
    fm_out[...] = fm
    xm = x_ref[...] * fm
    h1_out[...] = jnp.dot(xm, w1_ref[...], preferred_element_type=jnp.float32)


def _dense_c(dega_ref, degb_ref, h1_ref, dis_out, dis2_out, g1_out):
    dis = jax.lax.rsqrt(dega_ref[...] + degb_ref[...] + 1.0)
    dis_out[...] = dis
    dis2_out[...] = dis * dis
    g1_out[...] = dis * h1_ref[...]


def _dense_e(acc1_ref, dis_ref, dis2_ref, h1_ref, b1_ref, w2_ref,
             h2_out, g2_out):
    acc = acc1_ref[0] + acc1_ref[1]
    out1 = jax.nn.relu(dis_ref[...] * acc + dis2_ref[...] * h1_ref[...]
                       + b1_ref[...])
    h2 = jnp.dot(out1, w2_ref[...], preferred_element_type=jnp.float32)
    h2_out[...] = h2
    g2_out[...] = dis_ref[...] * h2


def _dense_g(acc2_ref, dis_ref, dis2_ref, h2_ref, b2_ref, out_ref):
    acc = acc2_ref[0] + acc2_ref[1]
    o = dis_ref[...] * acc + dis2_ref[...] * h2_ref[...] + b2_ref[...]
    mask = jax.lax.broadcasted_iota(jnp.int32, o.shape, 1) < 7
    neg = jnp.full_like(o, -jnp.inf)
    om = jnp.where(mask, o, neg)
    m = jnp.max(om, axis=1, keepdims=True)
    ex = jnp.where(mask, jnp.exp(o - m), jnp.zeros_like(o))
    lse = jnp.log(jnp.sum(ex, axis=1, keepdims=True))
    out_ref[...] = o - m - lse


def _row_spec(width):
    return pl.BlockSpec((ROW_BLK, width), lambda i: (i, 0))


def _acc_spec(width):
    return pl.BlockSpec((2, ROW_BLK, width), lambda i: (0, i, 0))


def _full_spec(shape):
    return pl.BlockSpec(shape, lambda i: tuple(0 for _ in shape))


def kernel(x, edge_index, edge_weight, feat_mask, W1, b1, W2, b2):
    row = edge_index[0].astype(jnp.int32)
    col = edge_index[1].astype(jnp.int32)
    w = edge_weight.astype(jnp.float32)

    pad = EP - E
    row2d = jnp.concatenate([row, jnp.zeros((pad,), jnp.int32)])
    col2d = jnp.concatenate([col, jnp.zeros((pad,), jnp.int32)])
    wp = jnp.concatenate([w, jnp.zeros((pad,), jnp.float32)])
    zeros1 = jnp.zeros((NP,), jnp.float32)
    zeros2 = jnp.zeros((NP, H), jnp.float32)

    grid = (N // ROW_BLK,)

    fm, h1 = pl.pallas_call(
        _dense_a,
        grid=grid,
        in_specs=[_row_spec(F_IN), _row_spec(F_IN), _full_spec((F_IN, H))],
        out_specs=[_row_spec(F_IN), _row_spec(H)],
        out_shape=[jax.ShapeDtypeStruct((N, F_IN), jnp.float32),
                   jax.ShapeDtypeStruct((N, H), jnp.float32)],
    )(x, feat_mask, W1)

    degp = _deg_call(col2d, wp, zeros1)
    dega = degp[0, :N, None]
    degb = degp[1, :N, None]

    dis, dis2, g1 = pl.pallas_call(
        _dense_c,
        grid=grid,
        in_specs=[_row_spec(1), _row_spec(1), _row_spec(H)],
        out_specs=[_row_spec(1), _row_spec(1), _row_spec(H)],
        out_shape=[jax.ShapeDtypeStruct((N, 1), jnp.float32),
                   jax.ShapeDtypeStruct((N, 1), jnp.float32),
                   jax.ShapeDtypeStruct((N, H), jnp.float32)],
    )(dega, degb, h1)

    acc1 = _prop_call(row2d, col2d, wp, g1, zeros2)[:, :N, :]

    W2p = jnp.zeros((H, H), jnp.float32).at[:, :W2.shape[1]].set(W2)
    b1r = b1[None, :]
    b2p = jnp.zeros((1, H), jnp.float32).at[0, :b2.shape[0]].set(b2)

    h2, g2 = pl.pallas_call(
        _dense_e,
        grid=grid,
        in_specs=[_acc_spec(H), _row_spec(1), _row_spec(1), _row_spec(H),
                  _full_spec((1, H)), _full_spec((H, H))],
        out_specs=[_row_spec(H), _row_spec(H)],
        out_shape=[jax.ShapeDtypeStruct((N, H), jnp.float32),
                   jax.ShapeDtypeStruct((N, H), jnp.float32)],
    )(acc1, dis, dis2, h1, b1r, W2p)

    acc2 = _prop_call(row2d, col2d, wp, g2, zeros2)[:, :N, :]

    outp = pl.pallas_call(
        _dense_g,
        grid=grid,
        in_specs=[_acc_spec(H), _row_spec(1), _row_spec(1), _row_spec(H),
                  _full_spec((1, H))],
        out_specs=_row_spec(H),
        out_shape=jax.ShapeDtypeStruct((N, H), jnp.float32),
    )(acc2, dis, dis2, h2, b2p)

    return outp[:, :7], fm


# speedup vs baseline: 45.2256x; 1.6320x over previous
import jax
import jax.numpy as jnp
from jax import lax
from jax.experimental import pallas as pl
from jax.experimental.pallas import tpu as pltpu
from jax.experimental.pallas import tpu_sc as plsc

N = 10000
F_IN = 128
H = 16
E = 320000
ROW_BLK = 2000

NC = 2
NS = 16
EPT = E // (NC * NS)
K = 2000
NCHUNK = EPT // K
NP = 10240
NPS = NP // NS


def _tile_out_copy(src_sh, dst_hbm, cid, sid):
    sl = pl.ds(sid * NPS, NPS)
    pltpu.sync_copy(src_sh.at[sl], dst_hbm.at[cid, sl])


def _tile_init_copy(zeros_hbm, dst_sh, sid):
    sl = pl.ds(sid * NPS, NPS)
    pltpu.sync_copy(zeros_hbm.at[sl], dst_sh.at[sl])


def _sc_deg_body(col_hbm, w_hbm, zeros_hbm, degp_hbm,
                 col_v0, col_v1, w_v0, w_v1, deg_sh, sem0, sem1):
    cid = lax.axis_index("c")
    sid = lax.axis_index("s")
    ebase = (cid * NS + sid) * EPT
    _tile_init_copy(zeros_hbm, deg_sh, sid)
    plsc.subcore_barrier()

    sems = (sem0, sem1)
    col_b = (col_v0, col_v1)
    w_b = (w_v0, w_v1)
    pltpu.sync_copy(col_hbm.at[pl.ds(ebase, K)], col_v0)
    pltpu.sync_copy(w_hbm.at[pl.ds(ebase, K)], w_v0)
    loads = [None, None]
    for i in range(NCHUNK):
        b = i % 2
        nb = 1 - b
        if i + 1 < NCHUNK:
            loads[nb] = (
                pltpu.async_copy(col_hbm.at[pl.ds(ebase + (i + 1) * K, K)],
                                 col_b[nb], sems[nb]),
                pltpu.async_copy(w_hbm.at[pl.ds(ebase + (i + 1) * K, K)],
                                 w_b[nb], sems[nb]),
            )
        pltpu.sync_copy(w_b[b], deg_sh.at[col_b[b]], add=True)
        if i + 1 < NCHUNK:
            loads[nb][0].wait()
            loads[nb][1].wait()

    plsc.subcore_barrier()
    _tile_out_copy(deg_sh, degp_hbm, cid, sid)


def _sc_prop_body(row_hbm, col_hbm, w_hbm, g_hbm, zeros_hbm, accp_hbm,
                  row_v0, row_v1, col_v0, col_v1, w_v0, w_v1,
                  msg_v0, msg_v1, acc_sh, gsem0, gsem1):
    cid = lax.axis_index("c")
    sid = lax.axis_index("s")
    ebase = (cid * NS + sid) * EPT
    _tile_init_copy(zeros_hbm, acc_sh, sid)
    plsc.subcore_barrier()

    gsems = (gsem0, gsem1)
    row_b = (row_v0, row_v1)
    col_b = (col_v0, col_v1)
    w_b = (w_v0, w_v1)
    msg_b = (msg_v0, msg_v1)

    def load_idx(i, b):
        pltpu.sync_copy(row_hbm.at[pl.ds(ebase + i * K, K)], row_b[b])
        pltpu.sync_copy(col_hbm.at[pl.ds(ebase + i * K, K)], col_b[b])
        pltpu.sync_copy(w_hbm.at[pl.ds(ebase + i * K, K)], w_b[b])

    load_idx(0, 0)
    gathers = [pltpu.async_copy(g_hbm.at[row_v0], msg_v0, gsems[0]),
               None]
    for i in range(NCHUNK):
        b = i % 2
        nb = 1 - b
        if i + 1 < NCHUNK:
            load_idx(i + 1, nb)
            gathers[nb] = pltpu.async_copy(g_hbm.at[row_b[nb]],
                                           msg_b[nb], gsems[nb])
        gathers[b].wait()

        mv = msg_b[b]
        wv_ref = w_b[b]

        def group(gi, c2):
            e0 = gi * 16
            wv = wv_ref[pl.ds(e0, 16)]
            for k in range(16):
                e = e0 + k
                mv[e] = mv[e] * wv[k]
            return c2

        lax.fori_loop(0, K // 16, group, 0)
        pltpu.sync_copy(mv, acc_sh.at[col_b[b]], add=True)

    plsc.subcore_barrier()
    _tile_out_copy(acc_sh, accp_hbm, cid, sid)


_SC_MESH = plsc.VectorSubcoreMesh(
    core_axis_name="c", subcore_axis_name="s", num_cores=NC, num_subcores=NS)

_deg_call = pl.kernel(
    _sc_deg_body,
    out_type=jax.ShapeDtypeStruct((NC, NP), jnp.float32),
    mesh=_SC_MESH,
    scratch_types=[
        pltpu.VMEM((K,), jnp.int32),
        pltpu.VMEM((K,), jnp.int32),
        pltpu.VMEM((K,), jnp.float32),
        pltpu.VMEM((K,), jnp.float32),
        pltpu.VMEM_SHARED((NP,), jnp.float32),
        pltpu.SemaphoreType.DMA,
        pltpu.SemaphoreType.DMA,
    ],
)

_prop_call = pl.kernel(
    _sc_prop_body,
    out_type=jax.ShapeDtypeStruct((NC, NP, H), jnp.float32),
    mesh=_SC_MESH,
    compiler_params=pltpu.CompilerParams(use_tc_tiling_on_sc=False),
    scratch_types=[
        pltpu.VMEM((K,), jnp.int32),
        pltpu.VMEM((K,), jnp.int32),
        pltpu.VMEM((K,), jnp.int32),
        pltpu.VMEM((K,), jnp.int32),
        pltpu.VMEM((K,), jnp.float32),
        pltpu.VMEM((K,), jnp.float32),
        pltpu.VMEM((K, H), jnp.float32),
        pltpu.VMEM((K, H), jnp.float32),
        pltpu.VMEM_SHARED((NP, H), jnp.float32),
        pltpu.SemaphoreType.DMA,
        pltpu.SemaphoreType.DMA,
    ],
)


def _dense_a(x_ref, fm_ref, w1_ref, fm_out, h1_out):
    fm = jax.nn.sigmoid(fm_ref[...])
    fm_out[...] = fm
    xm = x_ref[...] * fm
    h1_out[...] = jnp.dot(xm, w1_ref[...], preferred_element_type=jnp.float32)


def _dense_c(dega_ref, degb_ref, h1_ref, dis_out, dis2_out, g1_out):
    dis = jax.lax.rsqrt(dega_ref[...] + degb_ref[...] + 1.0)
    dis_out[...] = dis
    dis2_out[...] = dis * dis
    g1_out[...] = dis * h1_ref[...]


def _dense_e(acc1_ref, dis_ref, dis2_ref, h1_ref, b1_ref, w2_ref,
             h2_out, g2_out):
    acc = acc1_ref[0] + acc1_ref[1]
    out1 = jax.nn.relu(dis_ref[...] * acc + dis2_ref[...] * h1_ref[...]
                       + b1_ref[...])
    h2 = jnp.dot(out1, w2_ref[...], preferred_element_type=jnp.float32)
    h2_out[...] = h2
    g2_out[...] = dis_ref[...] * h2


def _dense_g(acc2_ref, dis_ref, dis2_ref, h2_ref, b2_ref, out_ref):
    acc = acc2_ref[0] + acc2_ref[1]
    o = dis_ref[...] * acc + dis2_ref[...] * h2_ref[...] + b2_ref[...]
    mask = jax.lax.broadcasted_iota(jnp.int32, o.shape, 1) < 7
    neg = jnp.full_like(o, -jnp.inf)
    om = jnp.where(mask, o, neg)
    m = jnp.max(om, axis=1, keepdims=True)
    ex = jnp.where(mask, jnp.exp(o - m), jnp.zeros_like(o))
    lse = jnp.log(jnp.sum(ex, axis=1, keepdims=True))
    out_ref[...] = o - m - lse


def _row_spec(width):
    return pl.BlockSpec((ROW_BLK, width), lambda i: (i, 0))


def _acc_spec(width):
    return pl.BlockSpec((2, ROW_BLK, width), lambda i: (0, i, 0))


def _full_spec(shape):
    return pl.BlockSpec(shape, lambda i: tuple(0 for _ in shape))


def kernel(x, edge_index, edge_weight, feat_mask, W1, b1, W2, b2):
    row = edge_index[0].astype(jnp.int32)
    col = edge_index[1].astype(jnp.int32)
    w = edge_weight.astype(jnp.float32)
    zeros1 = jnp.zeros((NP,), jnp.float32)
    zeros2 = jnp.zeros((NP, H), jnp.float32)

    grid = (N // ROW_BLK,)

    fm, h1 = pl.pallas_call(
        _dense_a,
        grid=grid,
        in_specs=[_row_spec(F_IN), _row_spec(F_IN), _full_spec((F_IN, H))],
        out_specs=[_row_spec(F_IN), _row_spec(H)],
        out_shape=[jax.ShapeDtypeStruct((N, F_IN), jnp.float32),
                   jax.ShapeDtypeStruct((N, H), jnp.float32)],
    )(x, feat_mask, W1)

    degp = _deg_call(col, w, zeros1)
    dega = degp[0, :N, None]
    degb = degp[1, :N, None]

    dis, dis2, g1 = pl.pallas_call(
        _dense_c,
        grid=grid,
        in_specs=[_row_spec(1), _row_spec(1), _row_spec(H)],
        out_specs=[_row_spec(1), _row_spec(1), _row_spec(H)],
        out_shape=[jax.ShapeDtypeStruct((N, 1), jnp.float32),
                   jax.ShapeDtypeStruct((N, 1), jnp.float32),
                   jax.ShapeDtypeStruct((N, H), jnp.float32)],
    )(dega, degb, h1)

    acc1 = _prop_call(row, col, w, g1, zeros2)[:, :N, :]

    W2p = jnp.zeros((H, H), jnp.float32).at[:, :W2.shape[1]].set(W2)
    b1r = b1[None, :]
    b2p = jnp.zeros((1, H), jnp.float32).at[0, :b2.shape[0]].set(b2)

    h2, g2 = pl.pallas_call(
        _dense_e,
        grid=grid,
        in_specs=[_acc_spec(H), _row_spec(1), _row_spec(1), _row_spec(H),
                  _full_spec((1, H)), _full_spec((H, H))],
        out_specs=[_row_spec(H), _row_spec(H)],
        out_shape=[jax.ShapeDtypeStruct((N, H), jnp.float32),
                   jax.ShapeDtypeStruct((N, H), jnp.float32)],
    )(acc1, dis, dis2, h1, b1r, W2p)

    acc2 = _prop_call(row, col, w, g2, zeros2)[:, :N, :]

    outp = pl.pallas_call(
        _dense_g,
        grid=grid,
        in_specs=[_acc_spec(H), _row_spec(1), _row_spec(1), _row_spec(H),
                  _full_spec((1, H))],
        out_specs=_row_spec(H),
        out_shape=jax.ShapeDtypeStruct((N, H), jnp.float32),
    )(acc2, dis, dis2, h2, b2p)

    return outp[:, :7], fm
